# baseline (speedup 1.0000x reference)
import functools

import jax
import jax.numpy as jnp
from jax import lax
from jax.experimental import pallas as pl
from jax.experimental.pallas import tpu as pltpu, tpu_sc as plsc

N = 10000
E = 160000
NCORE, NSUB, LANES = 2, 16, 16
NW = NCORE * NSUB
CHUNK = 128
EP_CH = 1280
NCHW = EP_CH // NW
NA = N + 128
RPT = 624
ZR = RPT + 16
EPS = 1e-5
F32 = jnp.float32


def _leaky(x):
    return jnp.where(x >= 0, x, 0.01 * x)


@functools.lru_cache(None)
def _scatter_fn(C):
    mesh = plsc.VectorSubcoreMesh(core_axis_name="c", subcore_axis_name="s")
    nbuf = 2

    @functools.partial(
        pl.kernel,
        out_type=jax.ShapeDtypeStruct((NCORE, N, C), F32),
        mesh=mesh,
        scratch_types=(
            [pltpu.VMEM((CHUNK,), jnp.int32) for _ in range(nbuf)]
            + [pltpu.VMEM((CHUNK,), jnp.int32) for _ in range(nbuf)]
            + [pltpu.VMEM((CHUNK, C), F32) for _ in range(nbuf)]
            + [pltpu.VMEM_SHARED((NA, C), F32)]
            + [pltpu.SemaphoreType.DMA for _ in range(2 * nbuf)]),
        compiler_params=pltpu.CompilerParams(use_tc_tiling_on_sc=False),
    )
    def k(u_hbm, src_hbm, dst_hbm, zero_hbm, out_hbm, *rest):
        sidx = rest[:nbuf]
        didx = rest[nbuf:2 * nbuf]
        rows = rest[2 * nbuf:3 * nbuf]
        acc = rest[3 * nbuf]
        isem = rest[3 * nbuf + 1:3 * nbuf + 1 + nbuf]
        gsem = rest[3 * nbuf + 1 + nbuf:]
        cid = lax.axis_index("c")
        sid = lax.axis_index("s")
        wid = sid * NCORE + cid
        base = sid * RPT
        row0 = wid * NCHW

        pltpu.sync_copy(zero_hbm.at[pl.ds(0, RPT)], acc.at[pl.ds(base, RPT)])

        @pl.when(sid == NSUB - 1)
        def _():
            pltpu.sync_copy(zero_hbm.at[pl.ds(0, N - NSUB * RPT)],
                            acc.at[pl.ds(NSUB * RPT, N - NSUB * RPT)])

        plsc.subcore_barrier()

        def body(i, carry):
            g = row0 + i * nbuf
            hi = [(pltpu.async_copy(src_hbm.at[g + b], sidx[b], isem[b]),
                   pltpu.async_copy(dst_hbm.at[g + b], didx[b], isem[b]))
                  for b in range(nbuf)]
            hg = []
            for b in range(nbuf):
                hi[b][0].wait()
                hi[b][1].wait()
                hg.append(pltpu.async_copy(u_hbm.at[sidx[b]], rows[b],
                                           gsem[b]))
            for b in range(nbuf):
                hg[b].wait()
                pltpu.sync_copy(rows[b], acc.at[didx[b]], add=True)
            return carry

        lax.fori_loop(0, NCHW // nbuf, body, 0)
        plsc.subcore_barrier()

        pltpu.sync_copy(acc.at[pl.ds(base, RPT)],
                        out_hbm.at[cid, pl.ds(base, RPT)])

        @pl.when(sid == NSUB - 1)
        def _():
            pltpu.sync_copy(acc.at[pl.ds(NSUB * RPT, 16)],
                            out_hbm.at[cid, pl.ds(NSUB * RPT, 16)])

    return k


def _scatter(u, src2, dst2):
    C = u.shape[1]
    return _scatter_fn(C)(u, src2, dst2, jnp.zeros((ZR, C), F32))


def _pre0_body(x_ref, degsc_ref, g_ref, b_ref, w_ref, u_ref, dinv_ref):
    x = x_ref[...]
    degs = degsc_ref[0] + degsc_ref[1]
    deg = degs[:, 0:1] + 1.0
    dinv = lax.rsqrt(jnp.maximum(deg, 1.0))
    m = jnp.mean(x)
    v = jnp.mean((x - m) ** 2)
    h = (x - m) / jnp.sqrt(v + EPS) * g_ref[0, 0] + b_ref[0, 0]
    hs = h * dinv
    u_ref[...] = hs * w_ref[0, :][None, :]
    dinv_ref[...] = dinv


def _post_block(p_ref, u_ref, dinv, b_ref, g_ref, be_ref):
    v = dinv * (p_ref[0] + p_ref[1] + u_ref[...]) + b_ref[...]
    m = jnp.mean(v, axis=0, keepdims=True)
    var = jnp.mean((v - m) ** 2, axis=0, keepdims=True)
    return _leaky((v - m) / jnp.sqrt(var + EPS) * g_ref[...] + be_ref[...])


def _mid_body(has_carry, nbo, cnb, *refs):
    (p_ref, u_ref, dinv_ref, b_ref, g_ref, be_ref, w_ref) = refs[0:7]
    carry = refs[7:7 + nbo] if has_carry else ()
    out_refs = refs[7 + nbo:] if has_carry else refs[7:]
    dinv = dinv_ref[...]
    h = _post_block(p_ref, u_ref, dinv, b_ref, g_ref, be_ref)
    acc = jnp.dot(h * dinv, w_ref[...], preferred_element_type=F32)
    for t in range(nbo):
        blk = acc[:, t * cnb:(t + 1) * cnb]
        if has_carry:
            blk = blk + carry[t][...]
        out_refs[t][...] = blk


def _pool_body(p_ref, u_ref, dinv_ref, b_ref, g_ref, be_ref, pool_ref):
    h = _post_block(p_ref, u_ref, dinv_ref[...], b_ref, g_ref, be_ref)
    pool_ref[...] = jnp.mean(h, axis=0, keepdims=True)


def _head_body(nbi, *refs):
    pools = refs[0:nbi]
    fcw_ref, fcb_ref, outw_ref, out_ref, lat_ref = refs[nbi:]
    pooled = jnp.concatenate([p[...] for p in pools], axis=1)
    pooled8 = jnp.broadcast_to(pooled, (8, pooled.shape[1]))
    lat = jnp.dot(pooled8, fcw_ref[...], preferred_element_type=F32)
    lat = lat + fcb_ref[...]
    lat_ref[...] = lat
    out_ref[...] = jnp.dot(lat, outw_ref[...], preferred_element_type=F32)


def kernel(input_batch, edge_index, params):
    pad = EP_CH * CHUNK - E
    src = jnp.concatenate(
        [edge_index[0], jnp.zeros((pad,), jnp.int32)]).reshape(EP_CH, CHUNK)
    dst = jnp.concatenate(
        [edge_index[1],
         N + (jnp.arange(pad, dtype=jnp.int32) % 128)]).reshape(EP_CH, CHUNK)

    convs = []
    for (W1, b1, g1, be1, W2, b2, g2, be2) in params['blocks']:
        convs.append((W1, b1, g1, be1))
        convs.append((W2, b2, g2, be2))
    widths = [w.shape[1] for (w, _, _, _) in convs]

    degsc = _scatter(jnp.ones((N, 16), F32), src, dst)

    W0 = convs[0][0]
    u0, dinv = pl.pallas_call(
        _pre0_body,
        out_shape=[jax.ShapeDtypeStruct((N, widths[0]), F32),
                   jax.ShapeDtypeStruct((N, 1), F32)],
    )(input_batch, degsc,
      params['bn0_g'].reshape(1, 1), params['bn0_b'].reshape(1, 1), W0)
    u_blocks = [u0]

    out8 = lat8 = None
    for i in range(10):
        C = widths[i]
        nbi = len(u_blocks)
        Cb = C // nbi
        p_blocks = [_scatter(ub, src, dst) for ub in u_blocks]
        (_, bi, gi, bei) = convs[i]
        bi = bi.reshape(1, C)
        gi = gi.reshape(1, C)
        bei = bei.reshape(1, C)
        if i < 9:
            Wn = convs[i + 1][0]
            Cn = widths[i + 1]
            nbo = Cn // 128 if Cn > 128 else 1
            cnb = Cn // nbo
            nxt = None
            for j in range(nbi):
                sl = slice(j * Cb, (j + 1) * Cb)
                body = functools.partial(_mid_body, nxt is not None, nbo, cnb)
                args = [p_blocks[j], u_blocks[j], dinv,
                        bi[:, sl], gi[:, sl], bei[:, sl], Wn[sl, :]]
                if nxt is not None:
                    args += list(nxt)
                nxt = pl.pallas_call(
                    body,
                    out_shape=[jax.ShapeDtypeStruct((N, cnb), F32)
                               for _ in range(nbo)],
                )(*args)
            u_blocks = list(nxt)
        else:
            pools = []
            for j in range(nbi):
                sl = slice(j * Cb, (j + 1) * Cb)
                pools.append(pl.pallas_call(
                    _pool_body,
                    out_shape=jax.ShapeDtypeStruct((1, Cb), F32),
                )(p_blocks[j], u_blocks[j], dinv,
                  bi[:, sl], gi[:, sl], bei[:, sl]))
            body = functools.partial(_head_body, nbi)
            out8, lat8 = pl.pallas_call(
                body,
                out_shape=[jax.ShapeDtypeStruct((8, N), F32),
                           jax.ShapeDtypeStruct((8, 128), F32)],
            )(*pools, params['fc_W'], params['fc_b'].reshape(1, 128),
              params['out_W'])
    return (out8[:1], lat8[:1])


# speedup vs baseline: 1.1077x; 1.1077x over previous
import functools

import jax
import jax.numpy as jnp
from jax import lax
from jax.experimental import pallas as pl
from jax.experimental.pallas import tpu as pltpu, tpu_sc as plsc

N = 10000
E = 160000
NCORE, NSUB, LANES = 2, 16, 16
NW = NCORE * NSUB
CHUNK = 128
EP_CH = 1280
NCHW = EP_CH // NW
NA = N + 128
RPT = 624
ZR = RPT + 16
EPS = 1e-5
F32 = jnp.float32


def _leaky(x):
    return jnp.where(x >= 0, x, 0.01 * x)


@functools.lru_cache(None)
def _scatter_fn(C):
    mesh = plsc.VectorSubcoreMesh(core_axis_name="c", subcore_axis_name="s")
    nbuf = 2

    @functools.partial(
        pl.kernel,
        out_type=jax.ShapeDtypeStruct((NCORE, N, C), F32),
        mesh=mesh,
        scratch_types=(
            [pltpu.VMEM((CHUNK,), jnp.int32) for _ in range(nbuf)]
            + [pltpu.VMEM((CHUNK,), jnp.int32) for _ in range(nbuf)]
            + [pltpu.VMEM((CHUNK, C), F32) for _ in range(nbuf)]
            + [pltpu.VMEM_SHARED((NA, C), F32)]
            + [pltpu.SemaphoreType.DMA for _ in range(2 * nbuf)]),
        compiler_params=pltpu.CompilerParams(use_tc_tiling_on_sc=False),
    )
    def k(u_hbm, src_hbm, dst_hbm, zero_hbm, out_hbm, *rest):
        sidx = rest[:nbuf]
        didx = rest[nbuf:2 * nbuf]
        rows = rest[2 * nbuf:3 * nbuf]
        acc = rest[3 * nbuf]
        isem = rest[3 * nbuf + 1:3 * nbuf + 1 + nbuf]
        gsem = rest[3 * nbuf + 1 + nbuf:]
        cid = lax.axis_index("c")
        sid = lax.axis_index("s")
        wid = sid * NCORE + cid
        base = sid * RPT
        row0 = wid * NCHW

        pltpu.sync_copy(zero_hbm.at[pl.ds(0, RPT)], acc.at[pl.ds(base, RPT)])

        @pl.when(sid == NSUB - 1)
        def _():
            pltpu.sync_copy(zero_hbm.at[pl.ds(0, N - NSUB * RPT)],
                            acc.at[pl.ds(NSUB * RPT, N - NSUB * RPT)])

        plsc.subcore_barrier()

        h0 = pltpu.async_copy(src_hbm.at[row0], sidx[0], isem[0])
        h1 = pltpu.async_copy(dst_hbm.at[row0], didx[0], isem[0])
        h0.wait()
        h1.wait()
        pltpu.async_copy(u_hbm.at[sidx[0]], rows[0], gsem[0])
        pltpu.async_copy(src_hbm.at[row0 + 1], sidx[1], isem[1])
        pltpu.async_copy(dst_hbm.at[row0 + 1], didx[1], isem[1])

        def body(g, carry):
            for b in range(2):
                nb = 1 - b
                ci = g * 2 + b
                r1 = row0 + lax.rem(ci + 1, NCHW)
                r2 = row0 + lax.rem(ci + 2, NCHW)
                pltpu.make_async_copy(src_hbm.at[r1], sidx[nb],
                                      isem[nb]).wait()
                pltpu.make_async_copy(dst_hbm.at[r1], didx[nb],
                                      isem[nb]).wait()
                pltpu.async_copy(u_hbm.at[sidx[nb]], rows[nb], gsem[nb])
                pltpu.make_async_copy(u_hbm.at[sidx[b]], rows[b],
                                      gsem[b]).wait()
                pltpu.sync_copy(rows[b], acc.at[didx[b]], add=True)
                pltpu.async_copy(src_hbm.at[r2], sidx[b], isem[b])
                pltpu.async_copy(dst_hbm.at[r2], didx[b], isem[b])
            return carry

        lax.fori_loop(0, NCHW // 2, body, 0)
        pltpu.make_async_copy(u_hbm.at[sidx[0]], rows[0], gsem[0]).wait()
        pltpu.make_async_copy(src_hbm.at[row0 + 1], sidx[1], isem[1]).wait()
        pltpu.make_async_copy(dst_hbm.at[row0 + 1], didx[1], isem[1]).wait()
        plsc.subcore_barrier()

        pltpu.sync_copy(acc.at[pl.ds(base, RPT)],
                        out_hbm.at[cid, pl.ds(base, RPT)])

        @pl.when(sid == NSUB - 1)
        def _():
            pltpu.sync_copy(acc.at[pl.ds(NSUB * RPT, 16)],
                            out_hbm.at[cid, pl.ds(NSUB * RPT, 16)])

    return k


def _scatter(u, src2, dst2):
    C = u.shape[1]
    return _scatter_fn(C)(u, src2, dst2, jnp.zeros((ZR, C), F32))


def _pre0_body(x_ref, degsc_ref, g_ref, b_ref, w_ref, u_ref, dinv_ref):
    x = x_ref[...]
    degs = degsc_ref[0] + degsc_ref[1]
    deg = degs[:, 0:1] + 1.0
    dinv = lax.rsqrt(jnp.maximum(deg, 1.0))
    m = jnp.mean(x)
    v = jnp.mean((x - m) ** 2)
    h = (x - m) / jnp.sqrt(v + EPS) * g_ref[0, 0] + b_ref[0, 0]
    hs = h * dinv
    u_ref[...] = hs * w_ref[0, :][None, :]
    dinv_ref[...] = dinv


def _post_block(p_ref, u_ref, dinv, b_ref, g_ref, be_ref):
    v = dinv * (p_ref[0] + p_ref[1] + u_ref[...]) + b_ref[...]
    m = jnp.mean(v, axis=0, keepdims=True)
    var = jnp.mean((v - m) ** 2, axis=0, keepdims=True)
    return _leaky((v - m) / jnp.sqrt(var + EPS) * g_ref[...] + be_ref[...])


def _mid_body(has_carry, nbo, cnb, *refs):
    (p_ref, u_ref, dinv_ref, b_ref, g_ref, be_ref, w_ref) = refs[0:7]
    carry = refs[7:7 + nbo] if has_carry else ()
    out_refs = refs[7 + nbo:] if has_carry else refs[7:]
    dinv = dinv_ref[...]
    h = _post_block(p_ref, u_ref, dinv, b_ref, g_ref, be_ref)
    acc = jnp.dot(h * dinv, w_ref[...], preferred_element_type=F32)
    for t in range(nbo):
        blk = acc[:, t * cnb:(t + 1) * cnb]
        if has_carry:
            blk = blk + carry[t][...]
        out_refs[t][...] = blk


def _pool_body(p_ref, u_ref, dinv_ref, b_ref, g_ref, be_ref, pool_ref):
    h = _post_block(p_ref, u_ref, dinv_ref[...], b_ref, g_ref, be_ref)
    pool_ref[...] = jnp.mean(h, axis=0, keepdims=True)


def _head_body(nbi, *refs):
    pools = refs[0:nbi]
    fcw_ref, fcb_ref, outw_ref, out_ref, lat_ref = refs[nbi:]
    pooled = jnp.concatenate([p[...] for p in pools], axis=1)
    pooled8 = jnp.broadcast_to(pooled, (8, pooled.shape[1]))
    lat = jnp.dot(pooled8, fcw_ref[...], preferred_element_type=F32)
    lat = lat + fcb_ref[...]
    lat_ref[...] = lat
    out_ref[...] = jnp.dot(lat, outw_ref[...], preferred_element_type=F32)


def kernel(input_batch, edge_index, params):
    pad = EP_CH * CHUNK - E
    src = jnp.concatenate(
        [edge_index[0], jnp.zeros((pad,), jnp.int32)]).reshape(EP_CH, CHUNK)
    dst = jnp.concatenate(
        [edge_index[1],
         N + (jnp.arange(pad, dtype=jnp.int32) % 128)]).reshape(EP_CH, CHUNK)

    convs = []
    for (W1, b1, g1, be1, W2, b2, g2, be2) in params['blocks']:
        convs.append((W1, b1, g1, be1))
        convs.append((W2, b2, g2, be2))
    widths = [w.shape[1] for (w, _, _, _) in convs]

    degsc = _scatter(jnp.ones((N, 16), F32), src, dst)

    W0 = convs[0][0]
    u0, dinv = pl.pallas_call(
        _pre0_body,
        out_shape=[jax.ShapeDtypeStruct((N, widths[0]), F32),
                   jax.ShapeDtypeStruct((N, 1), F32)],
    )(input_batch, degsc,
      params['bn0_g'].reshape(1, 1), params['bn0_b'].reshape(1, 1), W0)
    u_blocks = [u0]

    out8 = lat8 = None
    for i in range(10):
        C = widths[i]
        nbi = len(u_blocks)
        Cb = C // nbi
        p_blocks = [_scatter(ub, src, dst) for ub in u_blocks]
        (_, bi, gi, bei) = convs[i]
        bi = bi.reshape(1, C)
        gi = gi.reshape(1, C)
        bei = bei.reshape(1, C)
        if i < 9:
            Wn = convs[i + 1][0]
            Cn = widths[i + 1]
            nbo = Cn // 128 if Cn > 128 else 1
            cnb = Cn // nbo
            nxt = None
            for j in range(nbi):
                sl = slice(j * Cb, (j + 1) * Cb)
                body = functools.partial(_mid_body, nxt is not None, nbo, cnb)
                args = [p_blocks[j], u_blocks[j], dinv,
                        bi[:, sl], gi[:, sl], bei[:, sl], Wn[sl, :]]
                if nxt is not None:
                    args += list(nxt)
                nxt = pl.pallas_call(
                    body,
                    out_shape=[jax.ShapeDtypeStruct((N, cnb), F32)
                               for _ in range(nbo)],
                )(*args)
            u_blocks = list(nxt)
        else:
            pools = []
            for j in range(nbi):
                sl = slice(j * Cb, (j + 1) * Cb)
                pools.append(pl.pallas_call(
                    _pool_body,
                    out_shape=jax.ShapeDtypeStruct((1, Cb), F32),
                )(p_blocks[j], u_blocks[j], dinv,
                  bi[:, sl], gi[:, sl], bei[:, sl]))
            body = functools.partial(_head_body, nbi)
            out8, lat8 = pl.pallas_call(
                body,
                out_shape=[jax.ShapeDtypeStruct((8, N), F32),
                           jax.ShapeDtypeStruct((8, 128), F32)],
            )(*pools, params['fc_W'], params['fc_b'].reshape(1, 128),
              params['out_W'])
    return (out8[:1], lat8[:1])


# speedup vs baseline: 1.1911x; 1.0753x over previous
import functools

import jax
import jax.numpy as jnp
from jax import lax
from jax.experimental import pallas as pl
from jax.experimental.pallas import tpu as pltpu, tpu_sc as plsc

N = 10000
E = 160000
NCORE, NSUB, LANES = 2, 16, 16
NW = NCORE * NSUB
CHUNK = 128
EP_CH = 1280
NCHW = EP_CH // NW
NA = N + 128
RPT = 624
ZR = RPT + 16
EPS = 1e-5
F32 = jnp.float32


def _leaky(x):
    return jnp.where(x >= 0, x, 0.01 * x)


@functools.lru_cache(None)
def _scatter_fn(C):
    mesh = plsc.VectorSubcoreMesh(core_axis_name="c", subcore_axis_name="s")
    nbuf = 2

    @functools.partial(
        pl.kernel,
        out_type=jax.ShapeDtypeStruct((NCORE, N, C), F32),
        mesh=mesh,
        scratch_types=(
            [pltpu.VMEM((CHUNK,), jnp.int32) for _ in range(nbuf)]
            + [pltpu.VMEM((CHUNK,), jnp.int32) for _ in range(nbuf)]
            + [pltpu.VMEM((CHUNK, C), F32) for _ in range(nbuf)]
            + [pltpu.VMEM_SHARED((NA, C), F32)]
            + [pltpu.SemaphoreType.DMA for _ in range(2 * nbuf)]),
        compiler_params=pltpu.CompilerParams(use_tc_tiling_on_sc=False),
    )
    def k(u_hbm, src_hbm, dst_hbm, zero_hbm, out_hbm, *rest):
        sidx = rest[:nbuf]
        didx = rest[nbuf:2 * nbuf]
        rows = rest[2 * nbuf:3 * nbuf]
        acc = rest[3 * nbuf]
        isem = rest[3 * nbuf + 1:3 * nbuf + 1 + nbuf]
        gsem = rest[3 * nbuf + 1 + nbuf:]
        cid = lax.axis_index("c")
        sid = lax.axis_index("s")
        wid = sid * NCORE + cid
        base = sid * RPT

        pltpu.sync_copy(zero_hbm.at[pl.ds(0, RPT)], acc.at[pl.ds(base, RPT)])

        @pl.when(sid == NSUB - 1)
        def _():
            pltpu.sync_copy(zero_hbm.at[pl.ds(0, N - NSUB * RPT)],
                            acc.at[pl.ds(NSUB * RPT, N - NSUB * RPT)])

        plsc.subcore_barrier()

        h0 = pltpu.async_copy(src_hbm.at[wid], sidx[0], isem[0])
        h1 = pltpu.async_copy(dst_hbm.at[wid], didx[0], isem[0])
        h0.wait()
        h1.wait()
        pltpu.async_copy(u_hbm.at[sidx[0]], rows[0], gsem[0])
        pltpu.async_copy(src_hbm.at[NW + wid], sidx[1], isem[1])
        pltpu.async_copy(dst_hbm.at[NW + wid], didx[1], isem[1])

        def body(g, carry):
            for b in range(2):
                nb = 1 - b
                ci = g * 2 + b
                r1 = lax.rem(ci + 1, NCHW) * NW + wid
                r2 = lax.rem(ci + 2, NCHW) * NW + wid
                pltpu.make_async_copy(src_hbm.at[r1], sidx[nb],
                                      isem[nb]).wait()
                pltpu.make_async_copy(dst_hbm.at[r1], didx[nb],
                                      isem[nb]).wait()
                pltpu.async_copy(u_hbm.at[sidx[nb]], rows[nb], gsem[nb])
                pltpu.make_async_copy(u_hbm.at[sidx[b]], rows[b],
                                      gsem[b]).wait()
                pltpu.sync_copy(rows[b], acc.at[didx[b]], add=True)
                pltpu.async_copy(src_hbm.at[r2], sidx[b], isem[b])
                pltpu.async_copy(dst_hbm.at[r2], didx[b], isem[b])
            return carry

        lax.fori_loop(0, NCHW // 2, body, 0)
        pltpu.make_async_copy(u_hbm.at[sidx[0]], rows[0], gsem[0]).wait()
        pltpu.make_async_copy(src_hbm.at[NW + wid], sidx[1], isem[1]).wait()
        pltpu.make_async_copy(dst_hbm.at[NW + wid], didx[1], isem[1]).wait()
        plsc.subcore_barrier()

        pltpu.sync_copy(acc.at[pl.ds(base, RPT)],
                        out_hbm.at[cid, pl.ds(base, RPT)])

        @pl.when(sid == NSUB - 1)
        def _():
            pltpu.sync_copy(acc.at[pl.ds(NSUB * RPT, 16)],
                            out_hbm.at[cid, pl.ds(NSUB * RPT, 16)])

    return k


def _scatter(u, src2, dst2):
    C = u.shape[1]
    return _scatter_fn(C)(u, src2, dst2, jnp.zeros((ZR, C), F32))


def _pre0_body(x_ref, degsc_ref, g_ref, b_ref, w_ref, u_ref, dinv_ref):
    x = x_ref[...]
    degs = degsc_ref[0] + degsc_ref[1]
    deg = degs[:, 0:1] + 1.0
    dinv = lax.rsqrt(jnp.maximum(deg, 1.0))
    m = jnp.mean(x)
    v = jnp.mean((x - m) ** 2)
    h = (x - m) / jnp.sqrt(v + EPS) * g_ref[0, 0] + b_ref[0, 0]
    hs = h * dinv
    u_ref[...] = hs * w_ref[0, :][None, :]
    dinv_ref[...] = dinv


def _post_block(p_ref, u_ref, dinv, b_ref, g_ref, be_ref):
    v = dinv * (p_ref[0] + p_ref[1] + u_ref[...]) + b_ref[...]
    m = jnp.mean(v, axis=0, keepdims=True)
    var = jnp.mean((v - m) ** 2, axis=0, keepdims=True)
    return _leaky((v - m) / jnp.sqrt(var + EPS) * g_ref[...] + be_ref[...])


def _mid_body(has_carry, nbo, cnb, *refs):
    (p_ref, u_ref, dinv_ref, b_ref, g_ref, be_ref, w_ref) = refs[0:7]
    carry = refs[7:7 + nbo] if has_carry else ()
    out_refs = refs[7 + nbo:] if has_carry else refs[7:]
    dinv = dinv_ref[...]
    h = _post_block(p_ref, u_ref, dinv, b_ref, g_ref, be_ref)
    acc = jnp.dot(h * dinv, w_ref[...], preferred_element_type=F32)
    for t in range(nbo):
        blk = acc[:, t * cnb:(t + 1) * cnb]
        if has_carry:
            blk = blk + carry[t][...]
        out_refs[t][...] = blk


def _pool_body(p_ref, u_ref, dinv_ref, b_ref, g_ref, be_ref, pool_ref):
    h = _post_block(p_ref, u_ref, dinv_ref[...], b_ref, g_ref, be_ref)
    pool_ref[...] = jnp.mean(h, axis=0, keepdims=True)


def _head_body(nbi, *refs):
    pools = refs[0:nbi]
    fcw_ref, fcb_ref, outw_ref, out_ref, lat_ref = refs[nbi:]
    pooled = jnp.concatenate([p[...] for p in pools], axis=1)
    pooled8 = jnp.broadcast_to(pooled, (8, pooled.shape[1]))
    lat = jnp.dot(pooled8, fcw_ref[...], preferred_element_type=F32)
    lat = lat + fcb_ref[...]
    lat_ref[...] = lat
    out_ref[...] = jnp.dot(lat, outw_ref[...], preferred_element_type=F32)


def kernel(input_batch, edge_index, params):
    pad = EP_CH * CHUNK - E
    src = jnp.concatenate(
        [edge_index[0], jnp.zeros((pad,), jnp.int32)]).reshape(EP_CH, CHUNK)
    dst = jnp.concatenate(
        [edge_index[1],
         N + (jnp.arange(pad, dtype=jnp.int32) % 128)]).reshape(EP_CH, CHUNK)

    convs = []
    for (W1, b1, g1, be1, W2, b2, g2, be2) in params['blocks']:
        convs.append((W1, b1, g1, be1))
        convs.append((W2, b2, g2, be2))
    widths = [w.shape[1] for (w, _, _, _) in convs]

    degsc = _scatter(jnp.ones((N, 16), F32), src, dst)

    W0 = convs[0][0]
    u0, dinv = pl.pallas_call(
        _pre0_body,
        out_shape=[jax.ShapeDtypeStruct((N, widths[0]), F32),
                   jax.ShapeDtypeStruct((N, 1), F32)],
    )(input_batch, degsc,
      params['bn0_g'].reshape(1, 1), params['bn0_b'].reshape(1, 1), W0)
    u_blocks = [u0]

    out8 = lat8 = None
    for i in range(10):
        C = widths[i]
        nbi = len(u_blocks)
        Cb = C // nbi
        p_blocks = [_scatter(ub, src, dst) for ub in u_blocks]
        (_, bi, gi, bei) = convs[i]
        bi = bi.reshape(1, C)
        gi = gi.reshape(1, C)
        bei = bei.reshape(1, C)
        if i < 9:
            Wn = convs[i + 1][0]
            Cn = widths[i + 1]
            nbo = Cn // 128 if Cn > 128 else 1
            cnb = Cn // nbo
            nxt = None
            for j in range(nbi):
                sl = slice(j * Cb, (j + 1) * Cb)
                body = functools.partial(_mid_body, nxt is not None, nbo, cnb)
                args = [p_blocks[j], u_blocks[j], dinv,
                        bi[:, sl], gi[:, sl], bei[:, sl], Wn[sl, :]]
                if nxt is not None:
                    args += list(nxt)
                nxt = pl.pallas_call(
                    body,
                    out_shape=[jax.ShapeDtypeStruct((N, cnb), F32)
                               for _ in range(nbo)],
                )(*args)
            u_blocks = list(nxt)
        else:
            pools = []
            for j in range(nbi):
                sl = slice(j * Cb, (j + 1) * Cb)
                pools.append(pl.pallas_call(
                    _pool_body,
                    out_shape=jax.ShapeDtypeStruct((1, Cb), F32),
                )(p_blocks[j], u_blocks[j], dinv,
                  bi[:, sl], gi[:, sl], bei[:, sl]))
            body = functools.partial(_head_body, nbi)
            out8, lat8 = pl.pallas_call(
                body,
                out_shape=[jax.ShapeDtypeStruct((8, N), F32),
                           jax.ShapeDtypeStruct((8, 128), F32)],
            )(*pools, params['fc_W'], params['fc_b'].reshape(1, 128),
              params['out_W'])
    return (out8[:1], lat8[:1])
